# Initial kernel scaffold; baseline (speedup 1.0000x reference)
import functools

import jax
import jax.numpy as jnp
from jax import lax
from jax.experimental import pallas as pl
from jax.experimental.pallas import tpu as pltpu
from jax.experimental.pallas import tpu_sc as plsc

N_NODES = 10000
N_EDGES = 320000
N_REL = 8
NR = N_NODES * N_REL

NC = 2
NS = 16
LANES = 16

CHUNK = 128
NCHUNKS = N_EDGES // CHUNK
ROWS_PER_TILE = N_NODES // NS
CNT_SLICE = NR // NS

_SC_MESH = plsc.VectorSubcoreMesh(core_axis_name="c", subcore_axis_name="s")


def _prep_body(src_hbm, dst_hbm, typ_hbm, gkey_hbm, skey_hbm, inv_hbm,
               src_c, dst_c, typ_c, gkey_c, skey_c, ones_c, zbuf, cnt_sh):
    cid = lax.axis_index("c")
    sid = lax.axis_index("s")

    def zero_body(i, _):
        zbuf[pl.ds(i * LANES, LANES)] = jnp.zeros((LANES,), jnp.float32)
        return 0
    lax.fori_loop(0, (CNT_SLICE + LANES - 1) // LANES, zero_body, 0)
    pltpu.sync_copy(zbuf.at[pl.ds(0, CNT_SLICE)],
                    cnt_sh.at[pl.ds(sid * CNT_SLICE, CNT_SLICE)])
    for g in range(CHUNK // LANES):
        ones_c[pl.ds(g * LANES, LANES)] = jnp.ones((LANES,), jnp.float32)
    plsc.subcore_barrier()

    nt = 156 + jnp.where(sid < 4, 1, 0)

    @pl.when(cid == 0)
    def _():
        def count_body(t, _):
            j = sid + NS * t
            pltpu.sync_copy(dst_hbm.at[pl.ds(j * CHUNK, CHUNK)], dst_c)
            pltpu.sync_copy(typ_hbm.at[pl.ds(j * CHUNK, CHUNK)], typ_c)
            for g in range(CHUNK // LANES):
                sl = pl.ds(g * LANES, LANES)
                skey_c[sl] = dst_c[sl] * N_REL + typ_c[sl]
            pltpu.sync_copy(ones_c, cnt_sh.at[skey_c], add=True)
            return 0
        lax.fori_loop(0, nt, count_body, 0)

    @pl.when(cid == 1)
    def _():
        def key_body(t, _):
            j = sid + NS * t
            pltpu.sync_copy(src_hbm.at[pl.ds(j * CHUNK, CHUNK)], src_c)
            pltpu.sync_copy(dst_hbm.at[pl.ds(j * CHUNK, CHUNK)], dst_c)
            pltpu.sync_copy(typ_hbm.at[pl.ds(j * CHUNK, CHUNK)], typ_c)
            for g in range(CHUNK // LANES):
                sl = pl.ds(g * LANES, LANES)
                gkey_c[sl] = src_c[sl] * N_REL + typ_c[sl]
                skey_c[sl] = dst_c[sl] * N_REL + typ_c[sl]
            pltpu.sync_copy(gkey_c, gkey_hbm.at[pl.ds(j * CHUNK, CHUNK)])
            pltpu.sync_copy(skey_c, skey_hbm.at[pl.ds(j * CHUNK, CHUNK)])
            return 0
        lax.fori_loop(0, nt, key_body, 0)

    plsc.subcore_barrier()

    @pl.when(cid == 0)
    def _():
        pltpu.sync_copy(cnt_sh.at[pl.ds(sid * CNT_SLICE, CNT_SLICE)],
                        zbuf.at[pl.ds(0, CNT_SLICE)])
        def inv_body(i, _):
            sl = pl.ds(i * LANES, LANES)
            zbuf[sl] = 1.0 / jnp.maximum(zbuf[sl], 1.0)
            return 0
        lax.fori_loop(0, (CNT_SLICE + LANES - 1) // LANES, inv_body, 0)
        pltpu.sync_copy(zbuf.at[pl.ds(0, CNT_SLICE)],
                        inv_hbm.at[pl.ds(sid * CNT_SLICE, CNT_SLICE)])


_prep = pl.kernel(
    _prep_body,
    out_type=(
        jax.ShapeDtypeStruct((N_EDGES,), jnp.int32),
        jax.ShapeDtypeStruct((N_EDGES,), jnp.int32),
        jax.ShapeDtypeStruct((NR,), jnp.float32),
    ),
    mesh=_SC_MESH,
    scratch_types=[
        pltpu.VMEM((CHUNK,), jnp.int32),
        pltpu.VMEM((CHUNK,), jnp.int32),
        pltpu.VMEM((CHUNK,), jnp.int32),
        pltpu.VMEM((CHUNK,), jnp.int32),
        pltpu.VMEM((CHUNK,), jnp.int32),
        pltpu.VMEM((CHUNK,), jnp.float32),
        pltpu.VMEM((CNT_SLICE + 8,), jnp.float32),
        pltpu.VMEM_SHARED((NR,), jnp.float32),
    ],
)


def _scatter_body(d_out, y_hbm, gkey_hbm, skey_hbm, dstv_hbm, inv_hbm,
                  part_hbm, gkey_c, skey_c, dst_c, w_c, rows, acc, sem1, sem2):
    cid = lax.axis_index("c")
    sid = lax.axis_index("s")
    wid = sid * NC + cid

    zrows = ROWS_PER_TILE // 5
    def zero_body(i, _):
        for g in range(d_out // LANES):
            rows[i, pl.ds(g * LANES, LANES)] = jnp.zeros((LANES,), jnp.float32)
        return 0
    lax.fori_loop(0, zrows, zero_body, 0)
    base = sid * ROWS_PER_TILE
    for k in range(5):
        pltpu.sync_copy(rows.at[pl.ds(0, zrows)],
                        acc.at[pl.ds(base + k * zrows, zrows)])
    plsc.subcore_barrier()

    nt = 78 + jnp.where(wid < 4, 1, 0)

    def chunk_body(t, _):
        j = wid + NC * NS * t
        eb = j * CHUNK
        pltpu.sync_copy(gkey_hbm.at[pl.ds(eb, CHUNK)], gkey_c)
        pltpu.sync_copy(skey_hbm.at[pl.ds(eb, CHUNK)], skey_c)
        pltpu.sync_copy(dstv_hbm.at[pl.ds(eb, CHUNK)], dst_c)
        cp1 = pltpu.async_copy(y_hbm.at[gkey_c], rows, sem1)
        cp2 = pltpu.async_copy(inv_hbm.at[skey_c], w_c, sem2)
        cp1.wait()
        cp2.wait()

        def scale_body(i, _):
            wi = plsc.load_gather(w_c, [jnp.full((LANES,), i, jnp.int32)])
            for g in range(d_out // LANES):
                sl = pl.ds(g * LANES, LANES)
                rows[i, sl] = rows[i, sl] * wi
            return 0
        lax.fori_loop(0, CHUNK, scale_body, 0)

        pltpu.sync_copy(rows, acc.at[dst_c], add=True)
        return 0
    lax.fori_loop(0, nt, chunk_body, 0)

    plsc.subcore_barrier()
    pltpu.sync_copy(acc.at[pl.ds(base, ROWS_PER_TILE)],
                    part_hbm.at[cid, pl.ds(base, ROWS_PER_TILE)])


def _make_scatter(d_out):
    return pl.kernel(
        functools.partial(_scatter_body, d_out),
        out_type=jax.ShapeDtypeStruct((NC, N_NODES, d_out), jnp.float32),
        mesh=_SC_MESH,
        scratch_types=[
            pltpu.VMEM((CHUNK,), jnp.int32),
            pltpu.VMEM((CHUNK,), jnp.int32),
            pltpu.VMEM((CHUNK,), jnp.int32),
            pltpu.VMEM((CHUNK,), jnp.float32),
            pltpu.VMEM((CHUNK, d_out), jnp.float32),
            pltpu.VMEM_SHARED((N_NODES, d_out), jnp.float32),
            pltpu.SemaphoreType.DMA,
            pltpu.SemaphoreType.DMA,
        ],
    )


_scatter128 = _make_scatter(128)
_scatter64 = _make_scatter(64)


_BN = 2000


def _transform_body(x_ref, w_ref, root_ref, b_ref, y_ref, self_ref):
    x = x_ref[...]
    self_ref[...] = (
        jnp.dot(x, root_ref[...], preferred_element_type=jnp.float32)
        + b_ref[...]
    )
    for r in range(N_REL):
        y_ref[:, r, :] = jnp.dot(x, w_ref[r], preferred_element_type=jnp.float32)


def _transform(x, w, root, b):
    n, d_in = x.shape
    d_out = w.shape[2]
    y, self_out = pl.pallas_call(
        _transform_body,
        grid=(n // _BN,),
        in_specs=[
            pl.BlockSpec((_BN, d_in), lambda i: (i, 0)),
            pl.BlockSpec((N_REL, d_in, d_out), lambda i: (0, 0, 0)),
            pl.BlockSpec((d_in, d_out), lambda i: (0, 0)),
            pl.BlockSpec((1, d_out), lambda i: (0, 0)),
        ],
        out_specs=[
            pl.BlockSpec((_BN, N_REL, d_out), lambda i: (i, 0, 0)),
            pl.BlockSpec((_BN, d_out), lambda i: (i, 0)),
        ],
        out_shape=[
            jax.ShapeDtypeStruct((n, N_REL, d_out), jnp.float32),
            jax.ShapeDtypeStruct((n, d_out), jnp.float32),
        ],
    )(x, w, root, b.reshape(1, d_out))
    return y.reshape(n * N_REL, d_out), self_out


def _combine_body(s_ref, p0_ref, p1_ref, o_ref):
    s = s_ref[...] + p0_ref[...] + p1_ref[...]
    o_ref[...] = jnp.where(s > 0, s, jnp.expm1(s))


def _combine(self_out, parts):
    n, d = self_out.shape
    spec = pl.BlockSpec((_BN, d), lambda i: (i, 0))
    return pl.pallas_call(
        _combine_body,
        grid=(n // _BN,),
        in_specs=[spec, spec, spec],
        out_specs=spec,
        out_shape=jax.ShapeDtypeStruct((n, d), jnp.float32),
    )(self_out, parts[0], parts[1])


def kernel(x, edge_index, edge_type, W1, root1, b1, W2, root2, b2,
           W3, root3, b3):
    src = edge_index[0]
    dst = edge_index[1]
    gkey, skey, inv = _prep(src, dst, edge_type)

    def layer(h, w, root, b, scatter):
        y, self_out = _transform(h, w, root, b)
        parts = scatter(y, gkey, skey, dst, inv)
        return _combine(self_out, parts)

    h1 = layer(x, W1, root1, b1, _scatter128)
    h2 = layer(h1, W2, root2, b2, _scatter64)
    h3 = layer(h2, W3, root3, b3, _scatter64)
    return jnp.concatenate([h1, h2, h3], axis=1)


# speedup vs baseline: 16.4052x; 16.4052x over previous
import functools

import jax
import jax.numpy as jnp
from jax import lax
from jax.experimental import pallas as pl
from jax.experimental.pallas import tpu as pltpu
from jax.experimental.pallas import tpu_sc as plsc

N_NODES = 10000
N_EDGES = 320000
N_REL = 8
NR = N_NODES * N_REL

NC = 2
NS = 16
LANES = 16

CHUNK = 128
NCHUNKS = N_EDGES // CHUNK
N_PAD = 10240
ROWS_PER_TILE = N_PAD // NS
CNT_SLICE = NR // NS

_SC_MESH = plsc.VectorSubcoreMesh(core_axis_name="c", subcore_axis_name="s")


def _prep_body(src_hbm, dst_hbm, typ_hbm, gkey_hbm, skey_hbm, inv_hbm,
               src_c, dst_c, typ_c, gkey_c, skey_c, ones_c, zbuf, cnt_sh):
    cid = lax.axis_index("c")
    sid = lax.axis_index("s")

    def zero_body(i, _):
        zbuf[pl.ds(i * LANES, LANES)] = jnp.zeros((LANES,), jnp.float32)
        return 0
    lax.fori_loop(0, (CNT_SLICE + LANES - 1) // LANES, zero_body, 0)
    pltpu.sync_copy(zbuf.at[pl.ds(0, CNT_SLICE)],
                    cnt_sh.at[pl.ds(sid * CNT_SLICE, CNT_SLICE)])
    for g in range(CHUNK // LANES):
        ones_c[pl.ds(g * LANES, LANES)] = jnp.ones((LANES,), jnp.float32)
    plsc.subcore_barrier()

    nt = 156 + jnp.where(sid < 4, 1, 0)

    @pl.when(cid == 0)
    def _():
        def count_body(t, _):
            j = sid + NS * t
            pltpu.sync_copy(dst_hbm.at[pl.ds(j * CHUNK, CHUNK)], dst_c)
            pltpu.sync_copy(typ_hbm.at[pl.ds(j * CHUNK, CHUNK)], typ_c)
            for g in range(CHUNK // LANES):
                sl = pl.ds(g * LANES, LANES)
                skey_c[sl] = dst_c[sl] * N_REL + typ_c[sl]
            pltpu.sync_copy(ones_c, cnt_sh.at[skey_c], add=True)
            return 0
        lax.fori_loop(0, nt, count_body, 0)

    @pl.when(cid == 1)
    def _():
        def key_body(t, _):
            j = sid + NS * t
            pltpu.sync_copy(src_hbm.at[pl.ds(j * CHUNK, CHUNK)], src_c)
            pltpu.sync_copy(dst_hbm.at[pl.ds(j * CHUNK, CHUNK)], dst_c)
            pltpu.sync_copy(typ_hbm.at[pl.ds(j * CHUNK, CHUNK)], typ_c)
            for g in range(CHUNK // LANES):
                sl = pl.ds(g * LANES, LANES)
                gkey_c[sl] = src_c[sl] * N_REL + typ_c[sl]
                skey_c[sl] = dst_c[sl] * N_REL + typ_c[sl]
            pltpu.sync_copy(gkey_c, gkey_hbm.at[pl.ds(j * CHUNK, CHUNK)])
            pltpu.sync_copy(skey_c, skey_hbm.at[pl.ds(j * CHUNK, CHUNK)])
            return 0
        lax.fori_loop(0, nt, key_body, 0)

    plsc.subcore_barrier()

    @pl.when(cid == 0)
    def _():
        pltpu.sync_copy(cnt_sh.at[pl.ds(sid * CNT_SLICE, CNT_SLICE)],
                        zbuf.at[pl.ds(0, CNT_SLICE)])
        def inv_body(i, _):
            sl = pl.ds(i * LANES, LANES)
            zbuf[sl] = 1.0 / jnp.maximum(zbuf[sl], 1.0)
            return 0
        lax.fori_loop(0, (CNT_SLICE + LANES - 1) // LANES, inv_body, 0)
        pltpu.sync_copy(zbuf.at[pl.ds(0, CNT_SLICE)],
                        inv_hbm.at[pl.ds(sid * CNT_SLICE, CNT_SLICE)])


_prep = pl.kernel(
    _prep_body,
    out_type=(
        jax.ShapeDtypeStruct((N_EDGES,), jnp.int32),
        jax.ShapeDtypeStruct((N_EDGES,), jnp.int32),
        jax.ShapeDtypeStruct((NR,), jnp.float32),
    ),
    mesh=_SC_MESH,
    scratch_types=[
        pltpu.VMEM((CHUNK,), jnp.int32),
        pltpu.VMEM((CHUNK,), jnp.int32),
        pltpu.VMEM((CHUNK,), jnp.int32),
        pltpu.VMEM((CHUNK,), jnp.int32),
        pltpu.VMEM((CHUNK,), jnp.int32),
        pltpu.VMEM((CHUNK,), jnp.float32),
        pltpu.VMEM((CNT_SLICE + 8,), jnp.float32),
        pltpu.VMEM_SHARED((NR,), jnp.float32),
    ],
)


def _scatter_body(active, y_hbm, gkey_hbm, skey_hbm, dstv_hbm, inv_hbm,
                  part_hbm, gkey_c, skey_c, dst_c, w_c, rows, acc, sem1, sem2):
    cid = lax.axis_index("c")
    sid = lax.axis_index("s")
    wid = sid * NC + cid

    def zero_body(i, _):
        for g in range(128 // LANES):
            rows[i, pl.ds(g * LANES, LANES)] = jnp.zeros((LANES,), jnp.float32)
        return 0
    lax.fori_loop(0, CHUNK, zero_body, 0)
    base = sid * ROWS_PER_TILE
    for k in range(ROWS_PER_TILE // CHUNK):
        pltpu.sync_copy(rows, acc.at[pl.ds(base + k * CHUNK, CHUNK)])
    plsc.subcore_barrier()

    nt = 78 + jnp.where(wid < 4, 1, 0)

    def chunk_body(t, _):
        j = wid + NC * NS * t
        eb = j * CHUNK
        pltpu.sync_copy(gkey_hbm.at[pl.ds(eb, CHUNK)], gkey_c)
        pltpu.sync_copy(skey_hbm.at[pl.ds(eb, CHUNK)], skey_c)
        pltpu.sync_copy(dstv_hbm.at[pl.ds(eb, CHUNK)], dst_c)
        cp1 = pltpu.async_copy(y_hbm.at[gkey_c], rows, sem1)
        cp2 = pltpu.async_copy(inv_hbm.at[skey_c], w_c, sem2)
        cp1.wait()
        cp2.wait()

        def scale_body(g, _):
            wv = w_c[pl.ds(g * LANES, LANES)]
            for k in range(LANES):
                wi = wv[k]
                i = g * LANES + k
                for q in range(active // LANES):
                    sl = pl.ds(q * LANES, LANES)
                    rows[i, sl] = rows[i, sl] * wi
            return 0
        lax.fori_loop(0, CHUNK // LANES, scale_body, 0)

        pltpu.sync_copy(rows, acc.at[dst_c], add=True)
        return 0
    lax.fori_loop(0, nt, chunk_body, 0)

    plsc.subcore_barrier()
    pltpu.sync_copy(acc.at[pl.ds(base, ROWS_PER_TILE)],
                    part_hbm.at[cid, pl.ds(base, ROWS_PER_TILE)])


def _make_scatter(active):
    return pl.kernel(
        functools.partial(_scatter_body, active),
        out_type=jax.ShapeDtypeStruct((NC, N_PAD, 128), jnp.float32),
        mesh=_SC_MESH,
        scratch_types=[
            pltpu.VMEM((CHUNK,), jnp.int32),
            pltpu.VMEM((CHUNK,), jnp.int32),
            pltpu.VMEM((CHUNK,), jnp.int32),
            pltpu.VMEM((CHUNK,), jnp.float32),
            pltpu.VMEM((CHUNK, 128), jnp.float32),
            pltpu.VMEM_SHARED((N_PAD, 128), jnp.float32),
            pltpu.SemaphoreType.DMA,
            pltpu.SemaphoreType.DMA,
        ],
    )


_scatter128 = _make_scatter(128)
_scatter64 = _make_scatter(64)


_BN = 2000


def _transform_body(x_ref, w_ref, root_ref, b_ref, y_ref, self_ref):
    x = x_ref[...]
    self_ref[...] = (
        jnp.dot(x, root_ref[...], preferred_element_type=jnp.float32)
        + b_ref[...]
    )
    for r in range(N_REL):
        y_ref[:, r, :] = jnp.dot(x, w_ref[r], preferred_element_type=jnp.float32)


def _transform(x, w, root, b):
    n, d_in = x.shape
    d_y = w.shape[2]
    d_self = root.shape[1]
    y, self_out = pl.pallas_call(
        _transform_body,
        grid=(n // _BN,),
        in_specs=[
            pl.BlockSpec((_BN, d_in), lambda i: (i, 0)),
            pl.BlockSpec((N_REL, d_in, d_y), lambda i: (0, 0, 0)),
            pl.BlockSpec((d_in, d_self), lambda i: (0, 0)),
            pl.BlockSpec((1, d_self), lambda i: (0, 0)),
        ],
        out_specs=[
            pl.BlockSpec((_BN, N_REL, d_y), lambda i: (i, 0, 0)),
            pl.BlockSpec((_BN, d_self), lambda i: (i, 0)),
        ],
        out_shape=[
            jax.ShapeDtypeStruct((n, N_REL, d_y), jnp.float32),
            jax.ShapeDtypeStruct((n, d_self), jnp.float32),
        ],
    )(x, w, root, b.reshape(1, d_self))
    return y.reshape(n * N_REL, d_y), self_out


def _combine_body(s_ref, p0_ref, p1_ref, o_ref):
    s = s_ref[...] + p0_ref[...] + p1_ref[...]
    o_ref[...] = jnp.where(s > 0, s, jnp.exp(jnp.minimum(s, 0.0)) - 1.0)


def _combine(self_out, parts):
    n, d = self_out.shape
    spec = pl.BlockSpec((_BN, d), lambda i: (i, 0))
    return pl.pallas_call(
        _combine_body,
        grid=(n // _BN,),
        in_specs=[spec, spec, spec],
        out_specs=spec,
        out_shape=jax.ShapeDtypeStruct((n, d), jnp.float32),
    )(self_out, parts[0], parts[1])


def kernel(x, edge_index, edge_type, W1, root1, b1, W2, root2, b2,
           W3, root3, b3):
    src = edge_index[0]
    dst = edge_index[1]
    gkey, skey, inv = _prep(src, dst, edge_type)

    def layer(h, w, root, b, scatter):
        y, self_out = _transform(h, w, root, b)
        parts = scatter(y, gkey, skey, dst, inv)
        d = self_out.shape[1]
        return _combine(self_out, (parts[0, :N_NODES, :d], parts[1, :N_NODES, :d]))

    pad = ((0, 0), (0, 0), (0, 64))
    h1 = layer(x, W1, root1, b1, _scatter128)
    h2 = layer(h1, jnp.pad(W2, pad), root2, b2, _scatter64)
    h3 = layer(h2, jnp.pad(W3, pad), root3, b3, _scatter64)
    return jnp.concatenate([h1, h2, h3], axis=1)
